# Initial kernel scaffold; baseline (speedup 1.0000x reference)
import jax
import jax.numpy as jnp
from jax.experimental import pallas as pl


def kernel(x, edge_index, edge_attr, W_node, b_node, W_edge, b_edge):
    raise NotImplementedError("write your pallas kernel here")


# speedup vs baseline: 3.5492x; 3.5492x over previous
import functools

import jax
import jax.numpy as jnp
from jax import lax
from jax.experimental import pallas as pl
from jax.experimental.pallas import tpu as pltpu
from jax.experimental.pallas import tpu_sc as plsc

NC = 2
NS = 16
NW = NC * NS
L = 16
CHUNK = 128


def _sums_body(npad, cpw, rpt, d_in,
               x_hbm, row_hbm, col_hbm, sums_out,
               rowv, colv, rowsbuf, s_sums, sem):
    c = lax.axis_index("c")
    s = lax.axis_index("s")
    wid = s * NC + c

    zeros16 = jnp.zeros((L,), jnp.float32)
    dpl = d_in // L

    def f_rows(k, carry):
        rowsbuf[k // dpl, pl.ds((k % dpl) * L, L)] = zeros16
        return carry
    lax.fori_loop(0, CHUNK * dpl, f_rows, 0)

    def f_zero(r, carry):
        pltpu.sync_copy(rowsbuf, s_sums.at[pl.ds(s * rpt + r * CHUNK, CHUNK)])
        return carry
    lax.fori_loop(0, rpt // CHUNK, f_zero, 0)
    tail = rpt % CHUNK
    if tail:
        toff = s * rpt + (rpt // CHUNK) * CHUNK
        pltpu.sync_copy(rowsbuf.at[pl.ds(0, tail)], s_sums.at[pl.ds(toff, tail)])
    plsc.subcore_barrier()

    def step(t, carry):
        base = (wid * cpw + t) * CHUNK
        pltpu.sync_copy(row_hbm.at[pl.ds(base, CHUNK)], rowv)
        pltpu.sync_copy(col_hbm.at[pl.ds(base, CHUNK)], colv)
        pltpu.async_copy(x_hbm.at[colv], rowsbuf, sem).wait()
        pltpu.sync_copy(rowsbuf, s_sums.at[rowv], add=True)
        return carry
    lax.fori_loop(0, cpw, step, 0)
    plsc.subcore_barrier()

    off = s * rpt
    pltpu.sync_copy(s_sums.at[pl.ds(off, rpt)], sums_out.at[c, pl.ds(off, rpt)])


def _edge_body(npad, cpw, rpt, d_edge,
               row_hbm, eaf_hbm, a_out, cnt_out,
               rowv, eaf, zbuf, idxbuf, onesv, s_a, s_cnt):
    c = lax.axis_index("c")
    s = lax.axis_index("s")
    wid = s * NC + c

    zeros16 = jnp.zeros((L,), jnp.float32)
    ones16 = jnp.ones((L,), jnp.float32)
    fw = CHUNK * d_edge

    def f_zero_zbuf(k, carry):
        zbuf[pl.ds(k * L, L)] = zeros16
        return carry
    lax.fori_loop(0, fw // L, f_zero_zbuf, 0)

    def f_ones(k, carry):
        onesv[pl.ds(k * L, L)] = ones16
        return carry
    lax.fori_loop(0, CHUNK // L, f_ones, 0)

    base_a = s * (rpt * d_edge)
    nfull = (rpt * d_edge) // fw

    def f_za(r, carry):
        pltpu.sync_copy(zbuf, s_a.at[pl.ds(base_a + r * fw, fw)])
        return carry
    lax.fori_loop(0, nfull, f_za, 0)
    remw = rpt * d_edge - nfull * fw
    if remw:
        pltpu.sync_copy(zbuf.at[pl.ds(0, remw)],
                        s_a.at[pl.ds(base_a + nfull * fw, remw)])

    @pl.when(s == 0)
    def _():
        ncf = npad // fw

        def f_zc(r, carry):
            pltpu.sync_copy(zbuf, s_cnt.at[pl.ds(r * fw, fw)])
            return carry
        lax.fori_loop(0, ncf, f_zc, 0)
        remc = npad - ncf * fw
        if remc:
            pltpu.sync_copy(zbuf.at[pl.ds(0, remc)],
                            s_cnt.at[pl.ds(ncf * fw, remc)])

    plsc.subcore_barrier()

    def step(t, carry):
        base = (wid * cpw + t) * CHUNK
        pltpu.sync_copy(row_hbm.at[pl.ds(base, CHUNK)], rowv)
        pltpu.sync_copy(eaf_hbm.at[:, pl.ds(base, CHUNK)], eaf)
        for m in range(CHUNK // L):
            rv16 = rowv[pl.ds(L * m, L)] * d_edge
            for d in range(d_edge):
                idxbuf[d, pl.ds(L * m, L)] = rv16 + d
        for d in range(d_edge):
            pltpu.sync_copy(eaf.at[d], s_a.at[idxbuf.at[d]], add=True)
        pltpu.sync_copy(onesv, s_cnt.at[rowv], add=True)
        return carry
    lax.fori_loop(0, cpw, step, 0)
    plsc.subcore_barrier()

    pltpu.sync_copy(
        s_a.at[pl.ds(base_a, rpt * d_edge)],
        a_out.at[pl.ds(c * (npad * d_edge) + base_a, rpt * d_edge)])

    @pl.when(s == 0)
    def _():
        pltpu.sync_copy(s_cnt, cnt_out.at[pl.ds(c * npad, npad)])


def _combine_body(sums_ref, a_ref, cnt_ref, wn_ref, we_ref, b_ref, out_ref):
    s = sums_ref[0] + sums_ref[1]
    a = a_ref[0] + a_ref[1]
    cnt = cnt_ref[0] + cnt_ref[1]
    dn = (((1,), (1,)), ((), ()))
    num = lax.dot_general(s, wn_ref[...], dn, preferred_element_type=jnp.float32)
    num = num + lax.dot_general(a, we_ref[...], dn,
                                preferred_element_type=jnp.float32)
    num = num + cnt[:, None] * b_ref[...]
    out_ref[...] = num / jnp.maximum(cnt, 1.0)[:, None]


def kernel(x, edge_index, edge_attr, W_node, b_node, W_edge, b_edge):
    n, d_in = x.shape
    e = edge_index.shape[1]
    d_edge = edge_attr.shape[1]
    d_out = W_node.shape[0]

    npad = ((n + NS * 8 - 1) // (NS * 8)) * NS * 8
    if npad == n:
        npad += NS * 8
    rpt = npad // NS

    cpw = (e + NW * CHUNK - 1) // (NW * CHUNK)
    epad = cpw * NW * CHUNK
    pad = epad - e
    row = edge_index[0]
    col = edge_index[1]
    if pad:
        spare = max(npad - n, 1)
        pad_rows = n + (jnp.arange(pad, dtype=jnp.int32) % spare)
        row = jnp.concatenate([row, pad_rows])
        col = jnp.concatenate([col, jnp.zeros((pad,), jnp.int32)])
        edge_attr = jnp.concatenate(
            [edge_attr, jnp.zeros((pad, d_edge), edge_attr.dtype)])

    mesh = plsc.VectorSubcoreMesh(
        core_axis_name="c", subcore_axis_name="s",
        num_cores=NC, num_subcores=NS)
    f32 = jnp.float32

    sums2 = pl.kernel(
        functools.partial(_sums_body, npad, cpw, rpt, d_in),
        out_type=jax.ShapeDtypeStruct((NC, npad, d_in), f32),
        mesh=mesh,
        scratch_types=(
            pltpu.VMEM((CHUNK,), jnp.int32),
            pltpu.VMEM((CHUNK,), jnp.int32),
            pltpu.VMEM((CHUNK, d_in), f32),
            pltpu.VMEM_SHARED((npad, d_in), f32),
            pltpu.SemaphoreType.DMA,
        ),
        name="sc_gather_sums",
    )(x, row, col)

    a2, cnt2 = pl.kernel(
        functools.partial(_edge_body, npad, cpw, rpt, d_edge),
        out_type=(
            jax.ShapeDtypeStruct((NC * npad * d_edge,), f32),
            jax.ShapeDtypeStruct((NC * npad,), f32),
        ),
        mesh=mesh,
        scratch_types=(
            pltpu.VMEM((CHUNK,), jnp.int32),
            pltpu.VMEM((d_edge, CHUNK), f32),
            pltpu.VMEM((CHUNK * d_edge,), f32),
            pltpu.VMEM((d_edge, CHUNK), jnp.int32),
            pltpu.VMEM((CHUNK,), f32),
            pltpu.VMEM_SHARED((npad * d_edge,), f32),
            pltpu.VMEM_SHARED((npad,), f32),
        ),
        name="sc_edge_aggregate",
    )(row, edge_attr.T)
    a2 = a2.reshape(NC, npad, d_edge)
    cnt2 = cnt2.reshape(NC, npad)

    blk = 2048
    nblk = (n + blk - 1) // blk
    bias = (b_node + b_edge).reshape(1, d_out)
    out = pl.pallas_call(
        _combine_body,
        grid=(nblk,),
        in_specs=[
            pl.BlockSpec((NC, blk, d_in), lambda i: (0, i, 0)),
            pl.BlockSpec((NC, blk, d_edge), lambda i: (0, i, 0)),
            pl.BlockSpec((NC, blk), lambda i: (0, i)),
            pl.BlockSpec((d_out, d_in), lambda i: (0, 0)),
            pl.BlockSpec((d_out, d_edge), lambda i: (0, 0)),
            pl.BlockSpec((1, d_out), lambda i: (0, 0)),
        ],
        out_specs=pl.BlockSpec((blk, d_out), lambda i: (i, 0)),
        out_shape=jax.ShapeDtypeStruct((n, d_out), f32),
        name="tc_combine",
    )(sums2, a2, cnt2, W_node, W_edge, bias)
    return out


# speedup vs baseline: 4.1767x; 1.1768x over previous
import functools

import jax
import jax.numpy as jnp
from jax import lax
from jax.experimental import pallas as pl
from jax.experimental.pallas import tpu as pltpu
from jax.experimental.pallas import tpu_sc as plsc

NC = 2
NS = 16
NW = NC * NS
L = 16
CHUNK = 128


def _sums_body(npad, cpw, rpt, d_in,
               x_hbm, row_hbm, col_hbm, sums_out,
               rowv0, colv0, buf0, rowv1, colv1, buf1,
               s_sums, semg0, sems0, semg1, sems1):
    c = lax.axis_index("c")
    s = lax.axis_index("s")
    wid = s * NC + c

    zeros16 = jnp.zeros((L,), jnp.float32)
    dpl = d_in // L

    def f_rows(k, carry):
        buf0[k // dpl, pl.ds((k % dpl) * L, L)] = zeros16
        return carry
    lax.fori_loop(0, CHUNK * dpl, f_rows, 0)

    def f_zero(r, carry):
        pltpu.sync_copy(buf0, s_sums.at[pl.ds(s * rpt + r * CHUNK, CHUNK)])
        return carry
    lax.fori_loop(0, rpt // CHUNK, f_zero, 0)
    tail = rpt % CHUNK
    if tail:
        toff = s * rpt + (rpt // CHUNK) * CHUNK
        pltpu.sync_copy(buf0.at[pl.ds(0, tail)], s_sums.at[pl.ds(toff, tail)])
    plsc.subcore_barrier()

    slots = ((rowv0, colv0, buf0, semg0, sems0),
             (rowv1, colv1, buf1, semg1, sems1))

    def gather_desc(slot):
        rv, cv, buf, sg, ss = slot
        return pltpu.make_async_copy(x_hbm.at[cv], buf, sg)

    def scatter_desc(slot):
        rv, cv, buf, sg, ss = slot
        return pltpu.make_async_copy(buf, s_sums.at[rv], ss)

    def outer(so, carry):
        for b in (0, 1):
            slot = slots[b]
            rv, cv, buf, sg, ss = slot

            @pl.when(so > 0)
            def _():
                scatter_desc(slot).wait()

            base = (wid * cpw + so * 2 + b) * CHUNK
            pltpu.sync_copy(row_hbm.at[pl.ds(base, CHUNK)], rv)
            pltpu.sync_copy(col_hbm.at[pl.ds(base, CHUNK)], cv)
            gather_desc(slot).start()

            prev = slots[1 - b]
            if b == 0:
                @pl.when(so > 0)
                def _():
                    gather_desc(prev).wait()
                    scatter_desc(prev).start(add=True)
            else:
                gather_desc(prev).wait()
                scatter_desc(prev).start(add=True)
        return carry
    lax.fori_loop(0, cpw // 2, outer, 0)

    gather_desc(slots[1]).wait()
    scatter_desc(slots[1]).start(add=True)
    scatter_desc(slots[0]).wait()
    scatter_desc(slots[1]).wait()
    plsc.subcore_barrier()

    off = s * rpt
    pltpu.sync_copy(s_sums.at[pl.ds(off, rpt)], sums_out.at[c, pl.ds(off, rpt)])


def _edge_body(npad, cpw, rpt, d_edge,
               row_hbm, eaf_hbm, a_out, cnt_out,
               rowv0, eaf0, idx0, rowv1, eaf1, idx1,
               zbuf, onesv, s_a, s_cnt, semA0, semA1):
    c = lax.axis_index("c")
    s = lax.axis_index("s")
    wid = s * NC + c

    zeros16 = jnp.zeros((L,), jnp.float32)
    ones16 = jnp.ones((L,), jnp.float32)
    fw = CHUNK * d_edge

    def f_zero_zbuf(k, carry):
        zbuf[pl.ds(k * L, L)] = zeros16
        return carry
    lax.fori_loop(0, fw // L, f_zero_zbuf, 0)

    def f_ones(k, carry):
        onesv[pl.ds(k * L, L)] = ones16
        return carry
    lax.fori_loop(0, CHUNK // L, f_ones, 0)

    base_a = s * (rpt * d_edge)
    nfull = (rpt * d_edge) // fw

    def f_za(r, carry):
        pltpu.sync_copy(zbuf, s_a.at[pl.ds(base_a + r * fw, fw)])
        return carry
    lax.fori_loop(0, nfull, f_za, 0)
    remw = rpt * d_edge - nfull * fw
    if remw:
        pltpu.sync_copy(zbuf.at[pl.ds(0, remw)],
                        s_a.at[pl.ds(base_a + nfull * fw, remw)])

    @pl.when(s == 0)
    def _():
        ncf = npad // fw

        def f_zc(r, carry):
            pltpu.sync_copy(zbuf, s_cnt.at[pl.ds(r * fw, fw)])
            return carry
        lax.fori_loop(0, ncf, f_zc, 0)
        remc = npad - ncf * fw
        if remc:
            pltpu.sync_copy(zbuf.at[pl.ds(0, remc)],
                            s_cnt.at[pl.ds(ncf * fw, remc)])

    plsc.subcore_barrier()

    slots = ((rowv0, eaf0, idx0, semA0), (rowv1, eaf1, idx1, semA1))

    def descs(slot):
        rv, eaf, idxbuf, sem = slot
        d_list = [pltpu.make_async_copy(eaf.at[d], s_a.at[idxbuf.at[d]], sem)
                  for d in range(d_edge)]
        d_list.append(pltpu.make_async_copy(onesv, s_cnt.at[rv], sem))
        return d_list

    def load_chunk(slot, base):
        rv, eaf, idxbuf, sem = slot
        pltpu.sync_copy(row_hbm.at[pl.ds(base, CHUNK)], rv)
        pltpu.sync_copy(eaf_hbm.at[:, pl.ds(base, CHUNK)], eaf)
        for m in range(CHUNK // L):
            rv16 = rv[pl.ds(L * m, L)] * d_edge
            for d in range(d_edge):
                idxbuf[d, pl.ds(L * m, L)] = rv16 + d

    def fire(slot):
        for dsc in descs(slot):
            dsc.start(add=True)

    def drain(slot):
        for dsc in descs(slot):
            dsc.wait()

    def outer(so, carry):
        for b in (0, 1):
            slot = slots[b]
            base = (wid * cpw + so * 2 + b) * CHUNK
            load_chunk(slot, base)
            if b == 0:
                @pl.when(so > 0)
                def _():
                    drain(slots[1])
            else:
                drain(slots[0])
            fire(slot)
        return carry
    lax.fori_loop(0, cpw // 2, outer, 0)
    drain(slots[1])
    plsc.subcore_barrier()

    pltpu.sync_copy(
        s_a.at[pl.ds(base_a, rpt * d_edge)],
        a_out.at[pl.ds(c * (npad * d_edge) + base_a, rpt * d_edge)])

    @pl.when(s == 0)
    def _():
        pltpu.sync_copy(s_cnt, cnt_out.at[pl.ds(c * npad, npad)])


def _combine_body(sums_ref, a_ref, cnt_ref, wn_ref, we_ref, b_ref, out_ref):
    s = sums_ref[0] + sums_ref[1]
    a = a_ref[0] + a_ref[1]
    cnt = cnt_ref[0] + cnt_ref[1]
    dn = (((1,), (1,)), ((), ()))
    num = lax.dot_general(s, wn_ref[...], dn, preferred_element_type=jnp.float32)
    num = num + lax.dot_general(a, we_ref[...], dn,
                                preferred_element_type=jnp.float32)
    num = num + cnt[:, None] * b_ref[...]
    out_ref[...] = num / jnp.maximum(cnt, 1.0)[:, None]


def kernel(x, edge_index, edge_attr, W_node, b_node, W_edge, b_edge):
    n, d_in = x.shape
    e = edge_index.shape[1]
    d_edge = edge_attr.shape[1]
    d_out = W_node.shape[0]

    npad = ((n + NS * 8 - 1) // (NS * 8)) * NS * 8
    if npad == n:
        npad += NS * 8
    rpt = npad // NS

    cpw = (e + NW * CHUNK - 1) // (NW * CHUNK)
    cpw = max(2, cpw + (cpw % 2))
    epad = cpw * NW * CHUNK
    pad = epad - e
    row = edge_index[0]
    col = edge_index[1]
    if pad:
        spare = max(npad - n, 1)
        pad_rows = n + (jnp.arange(pad, dtype=jnp.int32) % spare)
        row = jnp.concatenate([row, pad_rows])
        col = jnp.concatenate([col, jnp.zeros((pad,), jnp.int32)])
        edge_attr = jnp.concatenate(
            [edge_attr, jnp.zeros((pad, d_edge), edge_attr.dtype)])

    mesh = plsc.VectorSubcoreMesh(
        core_axis_name="c", subcore_axis_name="s",
        num_cores=NC, num_subcores=NS)
    f32 = jnp.float32

    sums2 = pl.kernel(
        functools.partial(_sums_body, npad, cpw, rpt, d_in),
        out_type=jax.ShapeDtypeStruct((NC, npad, d_in), f32),
        mesh=mesh,
        scratch_types=(
            pltpu.VMEM((CHUNK,), jnp.int32),
            pltpu.VMEM((CHUNK,), jnp.int32),
            pltpu.VMEM((CHUNK, d_in), f32),
            pltpu.VMEM((CHUNK,), jnp.int32),
            pltpu.VMEM((CHUNK,), jnp.int32),
            pltpu.VMEM((CHUNK, d_in), f32),
            pltpu.VMEM_SHARED((npad, d_in), f32),
            pltpu.SemaphoreType.DMA,
            pltpu.SemaphoreType.DMA,
            pltpu.SemaphoreType.DMA,
            pltpu.SemaphoreType.DMA,
        ),
        name="sc_gather_sums",
    )(x, row, col)

    a2, cnt2 = pl.kernel(
        functools.partial(_edge_body, npad, cpw, rpt, d_edge),
        out_type=(
            jax.ShapeDtypeStruct((NC * npad * d_edge,), f32),
            jax.ShapeDtypeStruct((NC * npad,), f32),
        ),
        mesh=mesh,
        scratch_types=(
            pltpu.VMEM((CHUNK,), jnp.int32),
            pltpu.VMEM((d_edge, CHUNK), f32),
            pltpu.VMEM((d_edge, CHUNK), jnp.int32),
            pltpu.VMEM((CHUNK,), jnp.int32),
            pltpu.VMEM((d_edge, CHUNK), f32),
            pltpu.VMEM((d_edge, CHUNK), jnp.int32),
            pltpu.VMEM((CHUNK * d_edge,), f32),
            pltpu.VMEM((CHUNK,), f32),
            pltpu.VMEM_SHARED((npad * d_edge,), f32),
            pltpu.VMEM_SHARED((npad,), f32),
            pltpu.SemaphoreType.DMA,
            pltpu.SemaphoreType.DMA,
        ),
        name="sc_edge_aggregate",
    )(row, edge_attr.T)
    a2 = a2.reshape(NC, npad, d_edge)
    cnt2 = cnt2.reshape(NC, npad)

    blk = 2048
    nblk = (n + blk - 1) // blk
    bias = (b_node + b_edge).reshape(1, d_out)
    out = pl.pallas_call(
        _combine_body,
        grid=(nblk,),
        in_specs=[
            pl.BlockSpec((NC, blk, d_in), lambda i: (0, i, 0)),
            pl.BlockSpec((NC, blk, d_edge), lambda i: (0, i, 0)),
            pl.BlockSpec((NC, blk), lambda i: (0, i)),
            pl.BlockSpec((d_out, d_in), lambda i: (0, 0)),
            pl.BlockSpec((d_out, d_edge), lambda i: (0, 0)),
            pl.BlockSpec((1, d_out), lambda i: (0, 0)),
        ],
        out_specs=pl.BlockSpec((blk, d_out), lambda i: (i, 0)),
        out_shape=jax.ShapeDtypeStruct((n, d_out), f32),
        name="tc_combine",
    )(sums2, a2, cnt2, W_node, W_edge, bias)
    return out


# speedup vs baseline: 7.3774x; 1.7663x over previous
import functools

import jax
import jax.numpy as jnp
from jax import lax
from jax.experimental import pallas as pl
from jax.experimental.pallas import tpu as pltpu
from jax.experimental.pallas import tpu_sc as plsc

NC = 2
NS = 16
NW = NC * NS
L = 16
CHUNK = 128


def _sums_body(npad, cpw, rpt, d_in,
               x_hbm, row_hbm, col_hbm, sums_out,
               rowv0, colv0, buf0, rowv1, colv1, buf1,
               s_sums, semg0, sems0, semg1, sems1):
    c = lax.axis_index("c")
    s = lax.axis_index("s")
    wid = s * NC + c

    zeros16 = jnp.zeros((L,), jnp.float32)
    dpl = d_in // L

    def f_rows(k, carry):
        buf0[k // dpl, pl.ds((k % dpl) * L, L)] = zeros16
        return carry
    lax.fori_loop(0, CHUNK * dpl, f_rows, 0)

    def f_zero(r, carry):
        pltpu.sync_copy(buf0, s_sums.at[pl.ds(s * rpt + r * CHUNK, CHUNK)])
        return carry
    lax.fori_loop(0, rpt // CHUNK, f_zero, 0)
    tail = rpt % CHUNK
    if tail:
        toff = s * rpt + (rpt // CHUNK) * CHUNK
        pltpu.sync_copy(buf0.at[pl.ds(0, tail)], s_sums.at[pl.ds(toff, tail)])
    plsc.subcore_barrier()

    slots = ((rowv0, colv0, buf0, semg0, sems0),
             (rowv1, colv1, buf1, semg1, sems1))

    def gather_desc(slot):
        rv, cv, buf, sg, ss = slot
        return pltpu.make_async_copy(x_hbm.at[cv], buf, sg)

    def scatter_desc(slot):
        rv, cv, buf, sg, ss = slot
        return pltpu.make_async_copy(buf, s_sums.at[rv], ss)

    def outer(so, carry):
        for b in (0, 1):
            slot = slots[b]
            rv, cv, buf, sg, ss = slot

            @pl.when(so > 0)
            def _():
                scatter_desc(slot).wait()

            base = ((so * 2 + b) * NW + wid) * CHUNK
            pltpu.sync_copy(row_hbm.at[pl.ds(base, CHUNK)], rv)
            pltpu.sync_copy(col_hbm.at[pl.ds(base, CHUNK)], cv)
            gather_desc(slot).start()

            prev = slots[1 - b]
            if b == 0:
                @pl.when(so > 0)
                def _():
                    gather_desc(prev).wait()
                    scatter_desc(prev).start(add=True)
            else:
                gather_desc(prev).wait()
                scatter_desc(prev).start(add=True)
        return carry
    lax.fori_loop(0, cpw // 2, outer, 0)

    gather_desc(slots[1]).wait()
    scatter_desc(slots[1]).start(add=True)
    scatter_desc(slots[0]).wait()
    scatter_desc(slots[1]).wait()
    plsc.subcore_barrier()

    off = s * rpt
    pltpu.sync_copy(s_sums.at[pl.ds(off, rpt)], sums_out.at[c, pl.ds(off, rpt)])


def _edge_body(npad, cpw, rpt, d_edge,
               row_hbm, eaf_hbm, a_out, cnt_out,
               rowv0, eaf0, idx0, rowv1, eaf1, idx1,
               zbuf, onesv, s_a, s_cnt, semA0, semA1):
    c = lax.axis_index("c")
    s = lax.axis_index("s")
    wid = s * NC + c

    zeros16 = jnp.zeros((L,), jnp.float32)
    ones16 = jnp.ones((L,), jnp.float32)
    fw = CHUNK * d_edge

    def f_zero_zbuf(k, carry):
        zbuf[pl.ds(k * L, L)] = zeros16
        return carry
    lax.fori_loop(0, fw // L, f_zero_zbuf, 0)

    def f_ones(k, carry):
        onesv[pl.ds(k * L, L)] = ones16
        return carry
    lax.fori_loop(0, CHUNK // L, f_ones, 0)

    base_a = s * (rpt * d_edge)
    nfull = (rpt * d_edge) // fw

    def f_za(r, carry):
        pltpu.sync_copy(zbuf, s_a.at[pl.ds(base_a + r * fw, fw)])
        return carry
    lax.fori_loop(0, nfull, f_za, 0)
    remw = rpt * d_edge - nfull * fw
    if remw:
        pltpu.sync_copy(zbuf.at[pl.ds(0, remw)],
                        s_a.at[pl.ds(base_a + nfull * fw, remw)])

    @pl.when(s == 0)
    def _():
        ncf = npad // fw

        def f_zc(r, carry):
            pltpu.sync_copy(zbuf, s_cnt.at[pl.ds(r * fw, fw)])
            return carry
        lax.fori_loop(0, ncf, f_zc, 0)
        remc = npad - ncf * fw
        if remc:
            pltpu.sync_copy(zbuf.at[pl.ds(0, remc)],
                            s_cnt.at[pl.ds(ncf * fw, remc)])

    plsc.subcore_barrier()

    slots = ((rowv0, eaf0, idx0, semA0), (rowv1, eaf1, idx1, semA1))

    def descs(slot):
        rv, eaf, idxbuf, sem = slot
        d_list = [pltpu.make_async_copy(eaf.at[d], s_a.at[idxbuf.at[d]], sem)
                  for d in range(d_edge)]
        d_list.append(pltpu.make_async_copy(onesv, s_cnt.at[rv], sem))
        return d_list

    def load_chunk(slot, base):
        rv, eaf, idxbuf, sem = slot
        pltpu.sync_copy(row_hbm.at[pl.ds(base, CHUNK)], rv)
        pltpu.sync_copy(eaf_hbm.at[:, pl.ds(base, CHUNK)], eaf)
        for m in range(CHUNK // L):
            rv16 = rv[pl.ds(L * m, L)] * d_edge
            for d in range(d_edge):
                idxbuf[d, pl.ds(L * m, L)] = rv16 + d

    def fire(slot):
        for dsc in descs(slot):
            dsc.start(add=True)

    def drain(slot):
        for dsc in descs(slot):
            dsc.wait()

    def outer(so, carry):
        for b in (0, 1):
            slot = slots[b]
            base = ((so * 2 + b) * NW + wid) * CHUNK
            load_chunk(slot, base)
            if b == 0:
                @pl.when(so > 0)
                def _():
                    drain(slots[1])
            else:
                drain(slots[0])
            fire(slot)
        return carry
    lax.fori_loop(0, cpw // 2, outer, 0)
    drain(slots[1])
    plsc.subcore_barrier()

    pltpu.sync_copy(
        s_a.at[pl.ds(base_a, rpt * d_edge)],
        a_out.at[pl.ds(c * (npad * d_edge) + base_a, rpt * d_edge)])

    @pl.when(s == 0)
    def _():
        pltpu.sync_copy(s_cnt, cnt_out.at[pl.ds(c * npad, npad)])


def _combine_body(sums_ref, a_ref, cnt_ref, wn_ref, we_ref, b_ref, out_ref):
    s = sums_ref[0] + sums_ref[1]
    a = a_ref[0] + a_ref[1]
    cnt = cnt_ref[0] + cnt_ref[1]
    dn = (((1,), (1,)), ((), ()))
    num = lax.dot_general(s, wn_ref[...], dn, preferred_element_type=jnp.float32)
    num = num + lax.dot_general(a, we_ref[...], dn,
                                preferred_element_type=jnp.float32)
    num = num + cnt[:, None] * b_ref[...]
    out_ref[...] = num / jnp.maximum(cnt, 1.0)[:, None]


def kernel(x, edge_index, edge_attr, W_node, b_node, W_edge, b_edge):
    n, d_in = x.shape
    e = edge_index.shape[1]
    d_edge = edge_attr.shape[1]
    d_out = W_node.shape[0]

    npad = ((n + NS * 8 - 1) // (NS * 8)) * NS * 8
    if npad == n:
        npad += NS * 8
    rpt = npad // NS

    cpw = (e + NW * CHUNK - 1) // (NW * CHUNK)
    cpw = max(2, cpw + (cpw % 2))
    epad = cpw * NW * CHUNK
    pad = epad - e
    row = edge_index[0]
    col = edge_index[1]
    if pad:
        spare = max(npad - n, 1)
        ar = jnp.arange(pad, dtype=jnp.int32)
        row = jnp.concatenate([row, n + ar % spare])
        col = jnp.concatenate([col, (ar * 53) % n])
        edge_attr = jnp.concatenate(
            [edge_attr, jnp.zeros((pad, d_edge), edge_attr.dtype)])

    mesh = plsc.VectorSubcoreMesh(
        core_axis_name="c", subcore_axis_name="s",
        num_cores=NC, num_subcores=NS)
    f32 = jnp.float32

    sums2 = pl.kernel(
        functools.partial(_sums_body, npad, cpw, rpt, d_in),
        out_type=jax.ShapeDtypeStruct((NC, npad, d_in), f32),
        mesh=mesh,
        scratch_types=(
            pltpu.VMEM((CHUNK,), jnp.int32),
            pltpu.VMEM((CHUNK,), jnp.int32),
            pltpu.VMEM((CHUNK, d_in), f32),
            pltpu.VMEM((CHUNK,), jnp.int32),
            pltpu.VMEM((CHUNK,), jnp.int32),
            pltpu.VMEM((CHUNK, d_in), f32),
            pltpu.VMEM_SHARED((npad, d_in), f32),
            pltpu.SemaphoreType.DMA,
            pltpu.SemaphoreType.DMA,
            pltpu.SemaphoreType.DMA,
            pltpu.SemaphoreType.DMA,
        ),
        name="sc_gather_sums",
    )(x, row, col)

    a2, cnt2 = pl.kernel(
        functools.partial(_edge_body, npad, cpw, rpt, d_edge),
        out_type=(
            jax.ShapeDtypeStruct((NC * npad * d_edge,), f32),
            jax.ShapeDtypeStruct((NC * npad,), f32),
        ),
        mesh=mesh,
        scratch_types=(
            pltpu.VMEM((CHUNK,), jnp.int32),
            pltpu.VMEM((d_edge, CHUNK), f32),
            pltpu.VMEM((d_edge, CHUNK), jnp.int32),
            pltpu.VMEM((CHUNK,), jnp.int32),
            pltpu.VMEM((d_edge, CHUNK), f32),
            pltpu.VMEM((d_edge, CHUNK), jnp.int32),
            pltpu.VMEM((CHUNK * d_edge,), f32),
            pltpu.VMEM((CHUNK,), f32),
            pltpu.VMEM_SHARED((npad * d_edge,), f32),
            pltpu.VMEM_SHARED((npad,), f32),
            pltpu.SemaphoreType.DMA,
            pltpu.SemaphoreType.DMA,
        ),
        name="sc_edge_aggregate",
    )(row, edge_attr.T)
    a2 = a2.reshape(NC, npad, d_edge)
    cnt2 = cnt2.reshape(NC, npad)

    blk = 2048
    nblk = (n + blk - 1) // blk
    bias = (b_node + b_edge).reshape(1, d_out)
    out = pl.pallas_call(
        _combine_body,
        grid=(nblk,),
        in_specs=[
            pl.BlockSpec((NC, blk, d_in), lambda i: (0, i, 0)),
            pl.BlockSpec((NC, blk, d_edge), lambda i: (0, i, 0)),
            pl.BlockSpec((NC, blk), lambda i: (0, i)),
            pl.BlockSpec((d_out, d_in), lambda i: (0, 0)),
            pl.BlockSpec((d_out, d_edge), lambda i: (0, 0)),
            pl.BlockSpec((1, d_out), lambda i: (0, 0)),
        ],
        out_specs=pl.BlockSpec((blk, d_out), lambda i: (i, 0)),
        out_shape=jax.ShapeDtypeStruct((n, d_out), f32),
        name="tc_combine",
    )(sums2, a2, cnt2, W_node, W_edge, bias)
    return out


# speedup vs baseline: 9.4531x; 1.2814x over previous
import functools

import jax
import jax.numpy as jnp
from jax import lax
from jax.experimental import pallas as pl
from jax.experimental.pallas import tpu as pltpu
from jax.experimental.pallas import tpu_sc as plsc

NC = 2
NS = 16
NW = NC * NS
L = 16
CHUNK = 128


def _sums_body(npad, cpw, rpt, d_in,
               x_hbm, row_hbm, col_hbm, sums_out,
               slot_scratch, s_sums):
    c = lax.axis_index("c")
    s = lax.axis_index("s")
    wid = s * NC + c
    ring = len(slot_scratch)
    spw = cpw // ring

    zeros16 = jnp.zeros((L,), jnp.float32)
    dpl = d_in // L
    buf_init = slot_scratch[0][2]

    def f_rows(k, carry):
        buf_init[k // dpl, pl.ds((k % dpl) * L, L)] = zeros16
        return carry
    lax.fori_loop(0, CHUNK * dpl, f_rows, 0)

    def f_zero(r, carry):
        pltpu.sync_copy(buf_init, s_sums.at[pl.ds(s * rpt + r * CHUNK, CHUNK)])
        return carry
    lax.fori_loop(0, rpt // CHUNK, f_zero, 0)
    tail = rpt % CHUNK
    if tail:
        toff = s * rpt + (rpt // CHUNK) * CHUNK
        pltpu.sync_copy(buf_init.at[pl.ds(0, tail)],
                        s_sums.at[pl.ds(toff, tail)])
    plsc.subcore_barrier()

    def idx_descs(b, t):
        rv, cv, _, _, _, sl = slot_scratch[b]
        base = (t * NW + wid) * CHUNK
        return (pltpu.make_async_copy(row_hbm.at[pl.ds(base, CHUNK)], rv, sl),
                pltpu.make_async_copy(col_hbm.at[pl.ds(base, CHUNK)], cv, sl))

    def gather_desc(b):
        rv, cv, buf, sg, ss, sl = slot_scratch[b]
        return pltpu.make_async_copy(x_hbm.at[cv], buf, sg)

    def scatter_desc(b):
        rv, cv, buf, sg, ss, sl = slot_scratch[b]
        return pltpu.make_async_copy(buf, s_sums.at[rv], ss)

    for dsc in idx_descs(0, 0):
        dsc.start()

    def outer(so, carry):
        for b in range(ring):
            t = so * ring + b
            for dsc in idx_descs(b, t):
                dsc.wait()
            gather_desc(b).start()

            prevb = (b - 1) % ring
            if b == 0:
                @pl.when(so > 0)
                def _():
                    gather_desc(prevb).wait()
                    scatter_desc(prevb).start(add=True)
            else:
                gather_desc(prevb).wait()
                scatter_desc(prevb).start(add=True)

            nextb = (b + 1) % ring
            if b < ring - 2:
                @pl.when(so > 0)
                def _():
                    scatter_desc(nextb).wait()
                for dsc in idx_descs(nextb, t + 1):
                    dsc.start()
            elif b == ring - 2:
                @pl.when(so > 0)
                def _():
                    scatter_desc(nextb).wait()
                for dsc in idx_descs(nextb, t + 1):
                    dsc.start()
            else:
                scatter_desc(nextb).wait()

                @pl.when(so < spw - 1)
                def _():
                    for dsc in idx_descs(nextb, t + 1):
                        dsc.start()
        return carry
    lax.fori_loop(0, spw, outer, 0)

    gather_desc(ring - 1).wait()
    scatter_desc(ring - 1).start(add=True)
    scatter_desc(ring - 2).wait()
    scatter_desc(ring - 1).wait()
    plsc.subcore_barrier()

    off = s * rpt
    pltpu.sync_copy(s_sums.at[pl.ds(off, rpt)], sums_out.at[c, pl.ds(off, rpt)])


def _edge_body(npad, cpw, rpt, d_edge,
               row_hbm, eaf_hbm, a_out, cnt_out,
               rowv0, eaf0, idx0, semA0, semL0,
               rowv1, eaf1, idx1, semA1, semL1,
               zbuf, onesv, s_a, s_cnt):
    c = lax.axis_index("c")
    s = lax.axis_index("s")
    wid = s * NC + c

    zeros16 = jnp.zeros((L,), jnp.float32)
    ones16 = jnp.ones((L,), jnp.float32)
    fw = CHUNK * d_edge

    def f_zero_zbuf(k, carry):
        zbuf[pl.ds(k * L, L)] = zeros16
        return carry
    lax.fori_loop(0, fw // L, f_zero_zbuf, 0)

    def f_ones(k, carry):
        onesv[pl.ds(k * L, L)] = ones16
        return carry
    lax.fori_loop(0, CHUNK // L, f_ones, 0)

    base_a = s * (rpt * d_edge)
    nfull = (rpt * d_edge) // fw

    def f_za(r, carry):
        pltpu.sync_copy(zbuf, s_a.at[pl.ds(base_a + r * fw, fw)])
        return carry
    lax.fori_loop(0, nfull, f_za, 0)
    remw = rpt * d_edge - nfull * fw
    if remw:
        pltpu.sync_copy(zbuf.at[pl.ds(0, remw)],
                        s_a.at[pl.ds(base_a + nfull * fw, remw)])

    @pl.when(s == 0)
    def _():
        ncf = npad // fw

        def f_zc(r, carry):
            pltpu.sync_copy(zbuf, s_cnt.at[pl.ds(r * fw, fw)])
            return carry
        lax.fori_loop(0, ncf, f_zc, 0)
        remc = npad - ncf * fw
        if remc:
            pltpu.sync_copy(zbuf.at[pl.ds(0, remc)],
                            s_cnt.at[pl.ds(ncf * fw, remc)])

    plsc.subcore_barrier()

    slots = ((rowv0, eaf0, idx0, semA0, semL0), (rowv1, eaf1, idx1, semA1, semL1))
    spw = cpw // 2

    def load_descs(b, t):
        rv, eaf, idxbuf, sem, sl = slots[b]
        base = (t * NW + wid) * CHUNK
        return (pltpu.make_async_copy(row_hbm.at[pl.ds(base, CHUNK)], rv, sl),
                pltpu.make_async_copy(eaf_hbm.at[:, pl.ds(base, CHUNK)], eaf, sl))

    def descs(b):
        rv, eaf, idxbuf, sem, sl = slots[b]
        d_list = [pltpu.make_async_copy(eaf.at[d], s_a.at[idxbuf.at[d]], sem)
                  for d in range(d_edge)]
        d_list.append(pltpu.make_async_copy(onesv, s_cnt.at[rv], sem))
        return d_list

    for dsc in load_descs(0, 0):
        dsc.start()

    def outer(so, carry):
        for b in (0, 1):
            t = so * 2 + b
            rv, eaf, idxbuf, sem, sl = slots[b]
            for dsc in load_descs(b, t):
                dsc.wait()
            for m in range(CHUNK // L):
                rv16 = rv[pl.ds(L * m, L)] * d_edge
                for d in range(d_edge):
                    idxbuf[d, pl.ds(L * m, L)] = rv16 + d
            if b == 0:
                @pl.when(so > 0)
                def _():
                    for dsc in descs(1):
                        dsc.wait()
                for dsc in descs(0):
                    dsc.start(add=True)
                for dsc in load_descs(1, t + 1):
                    dsc.start()
            else:
                for dsc in descs(0):
                    dsc.wait()
                for dsc in descs(1):
                    dsc.start(add=True)

                @pl.when(so < spw - 1)
                def _():
                    for dsc in load_descs(0, t + 1):
                        dsc.start()
        return carry
    lax.fori_loop(0, spw, outer, 0)
    for dsc in descs(1):
        dsc.wait()
    plsc.subcore_barrier()

    pltpu.sync_copy(
        s_a.at[pl.ds(base_a, rpt * d_edge)],
        a_out.at[pl.ds(c * (npad * d_edge) + base_a, rpt * d_edge)])

    @pl.when(s == 0)
    def _():
        pltpu.sync_copy(s_cnt, cnt_out.at[pl.ds(c * npad, npad)])


def _combine_body(sums_ref, a_ref, cnt_ref, wn_ref, we_ref, b_ref, out_ref):
    s = sums_ref[0] + sums_ref[1]
    a = a_ref[0] + a_ref[1]
    cnt = cnt_ref[0] + cnt_ref[1]
    dn = (((1,), (1,)), ((), ()))
    num = lax.dot_general(s, wn_ref[...], dn, preferred_element_type=jnp.float32)
    num = num + lax.dot_general(a, we_ref[...], dn,
                                preferred_element_type=jnp.float32)
    num = num + cnt[:, None] * b_ref[...]
    out_ref[...] = num / jnp.maximum(cnt, 1.0)[:, None]


def kernel(x, edge_index, edge_attr, W_node, b_node, W_edge, b_edge):
    n, d_in = x.shape
    e = edge_index.shape[1]
    d_edge = edge_attr.shape[1]
    d_out = W_node.shape[0]

    npad = ((n + NS * 8 - 1) // (NS * 8)) * NS * 8
    if npad == n:
        npad += NS * 8
    rpt = npad // NS

    cpw = (e + NW * CHUNK - 1) // (NW * CHUNK)
    cpw = max(6, ((cpw + 5) // 6) * 6)
    epad = cpw * NW * CHUNK
    pad = epad - e
    row = edge_index[0]
    col = edge_index[1]
    if pad:
        spare = max(npad - n, 1)
        ar = jnp.arange(pad, dtype=jnp.int32)
        row = jnp.concatenate([row, n + ar % spare])
        col = jnp.concatenate([col, (ar * 53) % n])
        edge_attr = jnp.concatenate(
            [edge_attr, jnp.zeros((pad, d_edge), edge_attr.dtype)])

    mesh = plsc.VectorSubcoreMesh(
        core_axis_name="c", subcore_axis_name="s",
        num_cores=NC, num_subcores=NS)
    f32 = jnp.float32

    sums2 = pl.kernel(
        functools.partial(_sums_body, npad, cpw, rpt, d_in),
        out_type=jax.ShapeDtypeStruct((NC, npad, d_in), f32),
        mesh=mesh,
        scratch_types=(
            tuple(
                (pltpu.VMEM((CHUNK,), jnp.int32),
                 pltpu.VMEM((CHUNK,), jnp.int32),
                 pltpu.VMEM((CHUNK, d_in), f32),
                 pltpu.SemaphoreType.DMA,
                 pltpu.SemaphoreType.DMA,
                 pltpu.SemaphoreType.DMA)
                for _ in range(3)),
            pltpu.VMEM_SHARED((npad, d_in), f32),
        ),
        name="sc_gather_sums",
    )(x, row, col)

    a2, cnt2 = pl.kernel(
        functools.partial(_edge_body, npad, cpw, rpt, d_edge),
        out_type=(
            jax.ShapeDtypeStruct((NC * npad * d_edge,), f32),
            jax.ShapeDtypeStruct((NC * npad,), f32),
        ),
        mesh=mesh,
        scratch_types=(
            pltpu.VMEM((CHUNK,), jnp.int32),
            pltpu.VMEM((d_edge, CHUNK), f32),
            pltpu.VMEM((d_edge, CHUNK), jnp.int32),
            pltpu.SemaphoreType.DMA,
            pltpu.SemaphoreType.DMA,
            pltpu.VMEM((CHUNK,), jnp.int32),
            pltpu.VMEM((d_edge, CHUNK), f32),
            pltpu.VMEM((d_edge, CHUNK), jnp.int32),
            pltpu.SemaphoreType.DMA,
            pltpu.SemaphoreType.DMA,
            pltpu.VMEM((CHUNK * d_edge,), f32),
            pltpu.VMEM((CHUNK,), f32),
            pltpu.VMEM_SHARED((npad * d_edge,), f32),
            pltpu.VMEM_SHARED((npad,), f32),
        ),
        name="sc_edge_aggregate",
    )(row, edge_attr.T)
    a2 = a2.reshape(NC, npad, d_edge)
    cnt2 = cnt2.reshape(NC, npad)

    blk = 2048
    nblk = (n + blk - 1) // blk
    bias = (b_node + b_edge).reshape(1, d_out)
    out = pl.pallas_call(
        _combine_body,
        grid=(nblk,),
        in_specs=[
            pl.BlockSpec((NC, blk, d_in), lambda i: (0, i, 0)),
            pl.BlockSpec((NC, blk, d_edge), lambda i: (0, i, 0)),
            pl.BlockSpec((NC, blk), lambda i: (0, i)),
            pl.BlockSpec((d_out, d_in), lambda i: (0, 0)),
            pl.BlockSpec((d_out, d_edge), lambda i: (0, 0)),
            pl.BlockSpec((1, d_out), lambda i: (0, 0)),
        ],
        out_specs=pl.BlockSpec((blk, d_out), lambda i: (i, 0)),
        out_shape=jax.ShapeDtypeStruct((n, d_out), f32),
        name="tc_combine",
    )(sums2, a2, cnt2, W_node, W_edge, bias)
    return out


# speedup vs baseline: 9.4738x; 1.0022x over previous
import functools

import jax
import jax.numpy as jnp
from jax import lax
from jax.experimental import pallas as pl
from jax.experimental.pallas import tpu as pltpu
from jax.experimental.pallas import tpu_sc as plsc

NC = 2
NS = 16
NW = NC * NS
L = 16
CHUNK = 128


def _sums_body(npad, cpw, rpt, d_in,
               x_hbm, row_hbm, col_hbm, sums_out,
               slot_scratch, s_sums):
    c = lax.axis_index("c")
    s = lax.axis_index("s")
    wid = s * NC + c
    ring = len(slot_scratch)
    spw = cpw // ring

    zeros16 = jnp.zeros((L,), jnp.float32)
    dpl = d_in // L
    buf_init = slot_scratch[0][2]

    def f_rows(k, carry):
        buf_init[k // dpl, pl.ds((k % dpl) * L, L)] = zeros16
        return carry
    lax.fori_loop(0, CHUNK * dpl, f_rows, 0)

    def f_zero(r, carry):
        pltpu.sync_copy(buf_init, s_sums.at[pl.ds(s * rpt + r * CHUNK, CHUNK)])
        return carry
    lax.fori_loop(0, rpt // CHUNK, f_zero, 0)
    tail = rpt % CHUNK
    if tail:
        toff = s * rpt + (rpt // CHUNK) * CHUNK
        pltpu.sync_copy(buf_init.at[pl.ds(0, tail)],
                        s_sums.at[pl.ds(toff, tail)])
    plsc.subcore_barrier()

    def idx_descs(b, t):
        rv, cv, _, _, _, sl = slot_scratch[b]
        base = (t * NW + wid) * CHUNK
        return (pltpu.make_async_copy(row_hbm.at[pl.ds(base, CHUNK)], rv, sl),
                pltpu.make_async_copy(col_hbm.at[pl.ds(base, CHUNK)], cv, sl))

    def gather_desc(b):
        rv, cv, buf, sg, ss, sl = slot_scratch[b]
        return pltpu.make_async_copy(x_hbm.at[cv], buf, sg)

    def scatter_desc(b):
        rv, cv, buf, sg, ss, sl = slot_scratch[b]
        return pltpu.make_async_copy(buf, s_sums.at[rv], ss)

    for dsc in idx_descs(0, 0):
        dsc.start()

    def outer(so, carry):
        for b in range(ring):
            t = so * ring + b
            for dsc in idx_descs(b, t):
                dsc.wait()
            gather_desc(b).start()

            prevb = (b - 1) % ring
            if b == 0:
                @pl.when(so > 0)
                def _():
                    gather_desc(prevb).wait()
                    scatter_desc(prevb).start(add=True)
            else:
                gather_desc(prevb).wait()
                scatter_desc(prevb).start(add=True)

            nextb = (b + 1) % ring
            if b < ring - 2:
                @pl.when(so > 0)
                def _():
                    scatter_desc(nextb).wait()
                for dsc in idx_descs(nextb, t + 1):
                    dsc.start()
            elif b == ring - 2:
                @pl.when(so > 0)
                def _():
                    scatter_desc(nextb).wait()
                for dsc in idx_descs(nextb, t + 1):
                    dsc.start()
            else:
                scatter_desc(nextb).wait()

                @pl.when(so < spw - 1)
                def _():
                    for dsc in idx_descs(nextb, t + 1):
                        dsc.start()
        return carry
    lax.fori_loop(0, spw, outer, 0)

    gather_desc(ring - 1).wait()
    scatter_desc(ring - 1).start(add=True)
    scatter_desc(ring - 2).wait()
    scatter_desc(ring - 1).wait()
    plsc.subcore_barrier()

    off = s * rpt
    pltpu.sync_copy(s_sums.at[pl.ds(off, rpt)], sums_out.at[c, pl.ds(off, rpt)])


def _edge_body(npad, cpw, rpt, d_edge,
               row_hbm, eaf_hbm, a_out, cnt_out,
               rowv0, eaf0, idx0, semA0, semL0,
               rowv1, eaf1, idx1, semA1, semL1,
               rowv2, eaf2, idx2, semA2, semL2,
               zbuf, onesv, s_a, s_cnt):
    c = lax.axis_index("c")
    s = lax.axis_index("s")
    wid = s * NC + c

    zeros16 = jnp.zeros((L,), jnp.float32)
    ones16 = jnp.ones((L,), jnp.float32)
    fw = CHUNK * d_edge

    def f_zero_zbuf(k, carry):
        zbuf[pl.ds(k * L, L)] = zeros16
        return carry
    lax.fori_loop(0, fw // L, f_zero_zbuf, 0)

    def f_ones(k, carry):
        onesv[pl.ds(k * L, L)] = ones16
        return carry
    lax.fori_loop(0, CHUNK // L, f_ones, 0)

    base_a = s * (rpt * d_edge)
    nfull = (rpt * d_edge) // fw

    def f_za(r, carry):
        pltpu.sync_copy(zbuf, s_a.at[pl.ds(base_a + r * fw, fw)])
        return carry
    lax.fori_loop(0, nfull, f_za, 0)
    remw = rpt * d_edge - nfull * fw
    if remw:
        pltpu.sync_copy(zbuf.at[pl.ds(0, remw)],
                        s_a.at[pl.ds(base_a + nfull * fw, remw)])

    @pl.when(s == 0)
    def _():
        ncf = npad // fw

        def f_zc(r, carry):
            pltpu.sync_copy(zbuf, s_cnt.at[pl.ds(r * fw, fw)])
            return carry
        lax.fori_loop(0, ncf, f_zc, 0)
        remc = npad - ncf * fw
        if remc:
            pltpu.sync_copy(zbuf.at[pl.ds(0, remc)],
                            s_cnt.at[pl.ds(ncf * fw, remc)])

    plsc.subcore_barrier()

    slots = ((rowv0, eaf0, idx0, semA0, semL0),
             (rowv1, eaf1, idx1, semA1, semL1),
             (rowv2, eaf2, idx2, semA2, semL2))
    ring = len(slots)
    spw = cpw // ring

    def load_descs(b, t):
        rv, eaf, idxbuf, sem, sl = slots[b]
        base = (t * NW + wid) * CHUNK
        return (pltpu.make_async_copy(row_hbm.at[pl.ds(base, CHUNK)], rv, sl),
                pltpu.make_async_copy(eaf_hbm.at[:, pl.ds(base, CHUNK)], eaf, sl))

    def descs(b):
        rv, eaf, idxbuf, sem, sl = slots[b]
        d_list = [pltpu.make_async_copy(eaf.at[d], s_a.at[idxbuf.at[d]], sem)
                  for d in range(d_edge)]
        d_list.append(pltpu.make_async_copy(onesv, s_cnt.at[rv], sem))
        return d_list

    for dsc in load_descs(0, 0):
        dsc.start()

    def outer(so, carry):
        for b in range(ring):
            t = so * ring + b
            rv, eaf, idxbuf, sem, sl = slots[b]
            for dsc in load_descs(b, t):
                dsc.wait()
            for m in range(CHUNK // L):
                rv16 = rv[pl.ds(L * m, L)] * d_edge
                for d in range(d_edge):
                    idxbuf[d, pl.ds(L * m, L)] = rv16 + d
            drainb = (b + 1) % ring
            if b < ring - 1:
                @pl.when(so > 0)
                def _():
                    for dsc in descs(drainb):
                        dsc.wait()
                for dsc in descs(b):
                    dsc.start(add=True)
                for dsc in load_descs(drainb, t + 1):
                    dsc.start()
            else:
                for dsc in descs(drainb):
                    dsc.wait()
                for dsc in descs(b):
                    dsc.start(add=True)

                @pl.when(so < spw - 1)
                def _():
                    for dsc in load_descs(drainb, t + 1):
                        dsc.start()
        return carry
    lax.fori_loop(0, spw, outer, 0)
    for dsc in descs(ring - 2):
        dsc.wait()
    for dsc in descs(ring - 1):
        dsc.wait()
    plsc.subcore_barrier()

    pltpu.sync_copy(
        s_a.at[pl.ds(base_a, rpt * d_edge)],
        a_out.at[pl.ds(c * (npad * d_edge) + base_a, rpt * d_edge)])

    @pl.when(s == 0)
    def _():
        pltpu.sync_copy(s_cnt, cnt_out.at[pl.ds(c * npad, npad)])


def _combine_body(sums_ref, a_ref, cnt_ref, wn_ref, we_ref, b_ref, out_ref):
    s = sums_ref[0] + sums_ref[1]
    a = a_ref[0] + a_ref[1]
    cnt = cnt_ref[0] + cnt_ref[1]
    dn = (((1,), (1,)), ((), ()))
    num = lax.dot_general(s, wn_ref[...], dn, preferred_element_type=jnp.float32)
    num = num + lax.dot_general(a, we_ref[...], dn,
                                preferred_element_type=jnp.float32)
    num = num + cnt[:, None] * b_ref[...]
    out_ref[...] = num / jnp.maximum(cnt, 1.0)[:, None]


def kernel(x, edge_index, edge_attr, W_node, b_node, W_edge, b_edge):
    n, d_in = x.shape
    e = edge_index.shape[1]
    d_edge = edge_attr.shape[1]
    d_out = W_node.shape[0]

    npad = ((n + NS * 8 - 1) // (NS * 8)) * NS * 8
    if npad == n:
        npad += NS * 8
    rpt = npad // NS

    cpw = (e + NW * CHUNK - 1) // (NW * CHUNK)
    cpw = max(6, ((cpw + 5) // 6) * 6)
    epad = cpw * NW * CHUNK
    pad = epad - e
    row = edge_index[0]
    col = edge_index[1]
    if pad:
        spare = max(npad - n, 1)
        ar = jnp.arange(pad, dtype=jnp.int32)
        row = jnp.concatenate([row, n + ar % spare])
        col = jnp.concatenate([col, (ar * 53) % n])
        edge_attr = jnp.concatenate(
            [edge_attr, jnp.zeros((pad, d_edge), edge_attr.dtype)])

    mesh = plsc.VectorSubcoreMesh(
        core_axis_name="c", subcore_axis_name="s",
        num_cores=NC, num_subcores=NS)
    f32 = jnp.float32

    sums2 = pl.kernel(
        functools.partial(_sums_body, npad, cpw, rpt, d_in),
        out_type=jax.ShapeDtypeStruct((NC, npad, d_in), f32),
        mesh=mesh,
        scratch_types=(
            tuple(
                (pltpu.VMEM((CHUNK,), jnp.int32),
                 pltpu.VMEM((CHUNK,), jnp.int32),
                 pltpu.VMEM((CHUNK, d_in), f32),
                 pltpu.SemaphoreType.DMA,
                 pltpu.SemaphoreType.DMA,
                 pltpu.SemaphoreType.DMA)
                for _ in range(3)),
            pltpu.VMEM_SHARED((npad, d_in), f32),
        ),
        name="sc_gather_sums",
    )(x, row, col)

    a2, cnt2 = pl.kernel(
        functools.partial(_edge_body, npad, cpw, rpt, d_edge),
        out_type=(
            jax.ShapeDtypeStruct((NC * npad * d_edge,), f32),
            jax.ShapeDtypeStruct((NC * npad,), f32),
        ),
        mesh=mesh,
        scratch_types=(
            pltpu.VMEM((CHUNK,), jnp.int32),
            pltpu.VMEM((d_edge, CHUNK), f32),
            pltpu.VMEM((d_edge, CHUNK), jnp.int32),
            pltpu.SemaphoreType.DMA,
            pltpu.SemaphoreType.DMA,
            pltpu.VMEM((CHUNK,), jnp.int32),
            pltpu.VMEM((d_edge, CHUNK), f32),
            pltpu.VMEM((d_edge, CHUNK), jnp.int32),
            pltpu.SemaphoreType.DMA,
            pltpu.SemaphoreType.DMA,
            pltpu.VMEM((CHUNK,), jnp.int32),
            pltpu.VMEM((d_edge, CHUNK), f32),
            pltpu.VMEM((d_edge, CHUNK), jnp.int32),
            pltpu.SemaphoreType.DMA,
            pltpu.SemaphoreType.DMA,
            pltpu.VMEM((CHUNK * d_edge,), f32),
            pltpu.VMEM((CHUNK,), f32),
            pltpu.VMEM_SHARED((npad * d_edge,), f32),
            pltpu.VMEM_SHARED((npad,), f32),
        ),
        name="sc_edge_aggregate",
    )(row, edge_attr.T)
    a2 = a2.reshape(NC, npad, d_edge)
    cnt2 = cnt2.reshape(NC, npad)

    blk = 2048
    nblk = (n + blk - 1) // blk
    bias = (b_node + b_edge).reshape(1, d_out)
    out = pl.pallas_call(
        _combine_body,
        grid=(nblk,),
        in_specs=[
            pl.BlockSpec((NC, blk, d_in), lambda i: (0, i, 0)),
            pl.BlockSpec((NC, blk, d_edge), lambda i: (0, i, 0)),
            pl.BlockSpec((NC, blk), lambda i: (0, i)),
            pl.BlockSpec((d_out, d_in), lambda i: (0, 0)),
            pl.BlockSpec((d_out, d_edge), lambda i: (0, 0)),
            pl.BlockSpec((1, d_out), lambda i: (0, 0)),
        ],
        out_specs=pl.BlockSpec((blk, d_out), lambda i: (i, 0)),
        out_shape=jax.ShapeDtypeStruct((n, d_out), f32),
        name="tc_combine",
    )(sums2, a2, cnt2, W_node, W_edge, bias)
    return out
